# baseline (speedup 1.0000x reference)
import functools

import jax
import jax.numpy as jnp
from jax import lax
from jax.experimental import pallas as pl
from jax.experimental.pallas import tpu as pltpu
from jax.experimental.pallas import tpu_sc as plsc

_NC = 2
_NS = 16
_C = 128
_DH = 64
_CZ = 64
_NG = 3
_NSC = 2


@functools.lru_cache(maxsize=None)
def _make_prop(Np, ncw, n_layers):
    rpt = Np // _NS
    nz = rpt // _CZ
    ng = ncw // (_NG * _NSC)
    mesh = plsc.VectorSubcoreMesh(
        core_axis_name="c", subcore_axis_name="s",
        num_cores=_NC, num_subcores=_NS)
    tshape = jax.ShapeDtypeStruct((_NC * Np, _DH), jnp.float32)
    oshape = jax.ShapeDtypeStruct((_NC * Np, _DH), jnp.float32)

    @functools.partial(
        pl.kernel,
        out_type=(tshape, tshape, oshape),
        mesh=mesh,
        compiler_params=pltpu.CompilerParams(use_tc_tiling_on_sc=False),
        scratch_types=[
            pltpu.VMEM((ncw, _C), jnp.int32),
            pltpu.VMEM((_C,), jnp.int32),
            pltpu.VMEM((_C,), jnp.int32),
            pltpu.VMEM((_C,), jnp.int32),
            pltpu.VMEM((_C,), jnp.int32),
            pltpu.VMEM((_C,), jnp.int32),
            pltpu.VMEM((_C, _DH), jnp.float32),
            pltpu.VMEM((_C, _DH), jnp.float32),
            pltpu.VMEM((_C, _DH), jnp.float32),
            pltpu.VMEM((_C, _DH), jnp.float32),
            pltpu.VMEM((_C, _DH), jnp.float32),
            pltpu.VMEM((_C * 16,), jnp.float32),
            pltpu.VMEM((_C * 16,), jnp.float32),
            pltpu.VMEM((_CZ, _DH), jnp.float32),
            pltpu.VMEM((_CZ, _DH), jnp.float32),
            pltpu.VMEM_SHARED((Np, _DH), jnp.float32),
            pltpu.SemaphoreType.DMA,
            pltpu.SemaphoreType.DMA,
            pltpu.SemaphoreType.DMA,
            pltpu.SemaphoreType.DMA,
            pltpu.SemaphoreType.DMA,
            pltpu.SemaphoreType.DMA,
            pltpu.SemaphoreType.DMA,
        ],
    )
    def prop(tab0, accin, pidx_hbm, w_hbm, tabA, tabB, accout,
             pidx, c0, c1, c2, r0, r1, g0, g1, g2, s0, s1, w0, w1,
             a0, a1, acc, gs0, gs1, gs2, ss0, ss1, ws0, ws1):
        cid = lax.axis_index("c")
        sid = lax.axis_index("s")
        cols = (c0, c1, c2)
        rows = (r0, r1)
        gbuf = (g0, g1, g2)
        sbuf = (s0, s1)
        wbuf = (w0, w1)
        abuf = (a0, a1)
        gsem = (gs0, gs1, gs2)
        ssem = (ss0, ss1)
        wsem = (ws0, ws1)
        zero16 = jnp.zeros((16,), jnp.float32)

        pltpu.sync_copy(pidx_hbm.at[cid, sid], pidx)

        def _dec_col(ch, bg):
            for j in range(_C // 16):
                sl = pl.ds(j * 16, 16)
                cols[bg][sl] = lax.bitwise_and(pidx[ch, sl], 65535)

        def _dec_row(ch, bs):
            for j in range(_C // 16):
                sl = pl.ds(j * 16, 16)
                rows[bs][sl] = lax.shift_right_logical(pidx[ch, sl], 16)

        def _phase(src, accsrc, tab, scale):
            def _zrow(i, carry):
                for j in range(_DH // 16):
                    a0[i, pl.ds(j * 16, 16)] = zero16
                return carry

            lax.fori_loop(0, _CZ, _zrow, 0)
            for t in range(nz):
                pltpu.make_async_copy(
                    a0, acc.at[pl.ds(sid * rpt + t * _CZ, _CZ)], gs0).start()
            for t in range(nz):
                pltpu.make_async_copy(
                    a0, acc.at[pl.ds(sid * rpt + t * _CZ, _CZ)], gs0).wait()
            plsc.subcore_barrier()

            def _gather(ch, bg):
                return pltpu.make_async_copy(
                    src.at[cols[bg]], gbuf[bg], gsem[bg])

            def _wload(ch, bs):
                return pltpu.make_async_copy(
                    w_hbm.at[sid, ch], wbuf[bs], wsem[bs])

            def _scatter(ch, bs):
                return pltpu.make_async_copy(
                    sbuf[bs], acc.at[rows[bs]], ssem[bs])

            for ch in range(_NG):
                _dec_col(ch, ch % _NG)
                _gather(ch, ch % _NG).start()
            for ch in range(_NSC):
                _wload(ch, ch % _NSC).start()

            def _group(g, carry):
                for k in range(_NG * _NSC):
                    ch = g * (_NG * _NSC) + k
                    bg = k % _NG
                    bs = k % _NSC
                    _gather(ch, bg).wait()
                    _wload(ch, bs).wait()

                    if k < _NSC:
                        @pl.when(g > 0)
                        def _():
                            _scatter(ch - _NSC, bs).wait()
                    else:
                        _scatter(ch - _NSC, bs).wait()
                    _dec_row(ch, bs)

                    def _scale(e4, inner):
                        e = e4 * 4
                        wss = [wbuf[bs][pl.ds((e + kk) * 16, 16)]
                               for kk in range(4)]
                        for kk in range(4):
                            for j in range(_DH // 16):
                                sl = pl.ds(j * 16, 16)
                                sbuf[bs][e + kk, sl] = (
                                    gbuf[bg][e + kk, sl] * wss[kk])
                        return inner

                    lax.fori_loop(0, _C // 4, _scale, 0)
                    _scatter(ch, bs).start(add=True)

                    if k < _NG * _NSC - _NG:
                        _dec_col(ch + _NG, bg)
                        _gather(ch + _NG, bg).start()
                    else:
                        @pl.when(g < ng - 1)
                        def _():
                            _dec_col(ch + _NG, bg)
                            _gather(ch + _NG, bg).start()

                    if k < _NG * _NSC - _NSC:
                        _wload(ch + _NSC, bs).start()
                    else:
                        @pl.when(g < ng - 1)
                        def _():
                            _wload(ch + _NSC, bs).start()
                return carry

            lax.fori_loop(0, ng, _group, 0)
            for b in range(_NSC):
                _scatter(ncw - _NSC + b, b).wait()
            plsc.subcore_barrier()

            def _lsl(t):
                return pl.ds(sid * rpt + t * _CZ, _CZ)

            def _gsl(t):
                return pl.ds(cid * Np + sid * rpt + t * _CZ, _CZ)

            def _racc(t, b):
                return pltpu.make_async_copy(acc.at[_lsl(t)], abuf[b], gsem[b])

            def _rsrc(t, b):
                return pltpu.make_async_copy(
                    accsrc.at[_gsl(t)], sbuf[b].at[pl.ds(0, _CZ)], ssem[b])

            def _wacc(t, b):
                return pltpu.make_async_copy(
                    sbuf[b].at[pl.ds(0, _CZ)], accout.at[_gsl(t)], wsem[b])

            def _wtab(t, b):
                return pltpu.make_async_copy(abuf[b], tab.at[_gsl(t)], wsem[b])

            for t in range(min(2, nz)):
                _racc(t, t).start()
                _rsrc(t, t).start()
            for t in range(nz):
                b = t % 2
                _racc(t, b).wait()
                _rsrc(t, b).wait()

                def _acc_upd(i, carry):
                    for j in range(_DH // 16):
                        sl = pl.ds(j * 16, 16)
                        sbuf[b][i, sl] = (sbuf[b][i, sl] + abuf[b][i, sl]) * scale
                    return carry

                lax.fori_loop(0, _CZ, _acc_upd, 0)
                _wacc(t, b).start()
                if tab is not None:
                    _wtab(t, b).start()
                if t + 2 < nz:
                    _wacc(t, b).wait()
                    if tab is not None:
                        _wtab(t, b).wait()
                    _racc(t + 2, b).start()
                    _rsrc(t + 2, b).start()
            for t in range(max(0, nz - 2), nz):
                b = t % 2
                _wacc(t, b).wait()
                if tab is not None:
                    _wtab(t, b).wait()

        srcs = [tab0, tabA, tabB]
        for l in range(n_layers):
            _phase(
                srcs[l],
                accin if l == 0 else accout,
                srcs[l + 1] if l + 1 < n_layers else None,
                1.0 / (n_layers + 1) if l == n_layers - 1 else 1.0,
            )

    return prop


@functools.lru_cache(maxsize=None)
def _make_split(Np, D):
    B = 1024
    while Np % B:
        B //= 2

    def body(x_ref, o_ref):
        o_ref[0] = x_ref[:, :_DH]
        o_ref[1] = x_ref[:, _DH:]

    return pl.pallas_call(
        body,
        grid=(Np // B,),
        in_specs=[pl.BlockSpec((B, D), lambda i: (i, 0))],
        out_specs=pl.BlockSpec((2, B, _DH), lambda i: (0, i, 0)),
        out_shape=jax.ShapeDtypeStruct((2, Np, _DH), jnp.float32),
    )


@functools.lru_cache(maxsize=None)
def _make_merge(Np, D):
    B = 1024
    while Np % B:
        B //= 2

    def body(x_ref, o_ref):
        o_ref[:, :_DH] = x_ref[0]
        o_ref[:, _DH:] = x_ref[1]

    return pl.pallas_call(
        body,
        grid=(Np // B,),
        in_specs=[pl.BlockSpec((2, B, _DH), lambda i: (0, i, 0))],
        out_specs=pl.BlockSpec((B, D), lambda i: (i, 0)),
        out_shape=jax.ShapeDtypeStruct((Np, D), jnp.float32),
    )


def kernel(user_emb_weight, item_emb_weight, edge_index, edge_weight):
    U, D = user_emb_weight.shape
    N = U + item_emb_weight.shape[0]
    E = edge_weight.shape[0]

    all_emb = jnp.concatenate([user_emb_weight, item_emb_weight], axis=0)

    Np = -(-N // (_NS * _C)) * (_NS * _C)
    emb = all_emb
    if Np > N:
        emb = jnp.concatenate(
            [emb, jnp.zeros((Np - N, D), jnp.float32)], axis=0)

    ncw = -(-E // (_NS * _C))
    ncw += (-ncw) % (_NG * _NSC)
    Ep = _NS * _C * ncw
    row = edge_index[0].astype(jnp.int32)
    col = edge_index[1].astype(jnp.int32)
    w = edge_weight.astype(jnp.float32)
    if Ep > E:
        pad = Ep - E
        row = jnp.concatenate([row, jnp.zeros((pad,), jnp.int32)])
        col = jnp.concatenate([col, jnp.zeros((pad,), jnp.int32)])
        w = jnp.concatenate([w, jnp.zeros((pad,), jnp.float32)])
    col3 = col.reshape(_NS, ncw, _C)
    row3 = row.reshape(_NS, ncw, _C)
    rsh = jnp.left_shift(row3, 16)
    pidx4 = jnp.stack([
        jnp.bitwise_or(col3, rsh),
        jnp.bitwise_or(col3 + Np, rsh),
    ])
    wexp = jnp.repeat(w, 16).reshape(_NS, ncw, _C * 16)

    emb2 = _make_split(Np, D)(emb).reshape(_NC * Np, _DH)
    _, _, acc = _make_prop(Np, ncw, 3)(emb2, emb2, pidx4, wexp)
    out = _make_merge(Np, D)(acc.reshape(_NC, Np, _DH))[:N]
    return out[:U], out[U:]


# speedup vs baseline: 1.7512x; 1.7512x over previous
import functools

import jax
import jax.numpy as jnp
from jax import lax
from jax.experimental import pallas as pl
from jax.experimental.pallas import tpu as pltpu
from jax.experimental.pallas import tpu_sc as plsc

_NC = 2
_NS = 16
_C = 128
_DH = 64
_CZ = 64


@functools.lru_cache(maxsize=None)
def _make_prop(Np, ncw, n_layers):
    rpt = Np // _NS
    nz = rpt // _CZ
    mesh = plsc.VectorSubcoreMesh(
        core_axis_name="c", subcore_axis_name="s",
        num_cores=_NC, num_subcores=_NS)
    tshape = jax.ShapeDtypeStruct((_NC * Np, _DH), jnp.float32)
    oshape = jax.ShapeDtypeStruct((_NC * Np, _DH), jnp.float32)

    @functools.partial(
        pl.kernel,
        out_type=(tshape, tshape, oshape),
        mesh=mesh,
        compiler_params=pltpu.CompilerParams(use_tc_tiling_on_sc=False),
        scratch_types=[
            pltpu.VMEM((ncw, _C), jnp.int32),
            pltpu.VMEM((ncw, _C), jnp.int32),
            pltpu.VMEM((_C, _DH), jnp.float32),
            pltpu.VMEM((_C, _DH), jnp.float32),
            pltpu.VMEM((_C, _DH), jnp.float32),
            pltpu.VMEM((_C, _DH), jnp.float32),
            pltpu.VMEM((_C * 16,), jnp.float32),
            pltpu.VMEM((_C * 16,), jnp.float32),
            pltpu.VMEM((_CZ, _DH), jnp.float32),
            pltpu.VMEM((_CZ, _DH), jnp.float32),
            pltpu.VMEM_SHARED((Np, _DH), jnp.float32),
            pltpu.SemaphoreType.DMA,
            pltpu.SemaphoreType.DMA,
            pltpu.SemaphoreType.DMA,
            pltpu.SemaphoreType.DMA,
            pltpu.SemaphoreType.DMA,
            pltpu.SemaphoreType.DMA,
        ],
    )
    def prop(tab0, accin, col_hbm, row_hbm, w_hbm, tabA, tabB, accout,
             colv, rowv, g0, g1, s0, s1, w0, w1, a0, a1, acc,
             gs0, gs1, ss0, ss1, ws0, ws1):
        cid = lax.axis_index("c")
        sid = lax.axis_index("s")
        gbuf = (g0, g1)
        sbuf = (s0, s1)
        wbuf = (w0, w1)
        abuf = (a0, a1)
        gsem = (gs0, gs1)
        ssem = (ss0, ss1)
        wsem = (ws0, ws1)
        zero16 = jnp.zeros((16,), jnp.float32)

        pltpu.sync_copy(col_hbm.at[cid, sid], colv)
        pltpu.sync_copy(row_hbm.at[sid], rowv)

        def _phase(src, accsrc, tab, scale):
            def _zrow(i, carry):
                for j in range(_DH // 16):
                    a0[i, pl.ds(j * 16, 16)] = zero16
                return carry

            lax.fori_loop(0, _CZ, _zrow, 0)
            for t in range(nz):
                pltpu.make_async_copy(
                    a0, acc.at[pl.ds(sid * rpt + t * _CZ, _CZ)], gs0).start()
            for t in range(nz):
                pltpu.make_async_copy(
                    a0, acc.at[pl.ds(sid * rpt + t * _CZ, _CZ)], gs0).wait()
            plsc.subcore_barrier()

            def _gather(ch, b):
                return pltpu.make_async_copy(
                    src.at[colv.at[ch]], gbuf[b], gsem[b])

            def _wload(ch, b):
                return pltpu.make_async_copy(
                    w_hbm.at[sid, ch], wbuf[b], wsem[b])

            def _scatter(ch, b):
                return pltpu.make_async_copy(
                    sbuf[b], acc.at[rowv.at[ch]], ssem[b])

            for b in range(2):
                _gather(b, b).start()
                _wload(b, b).start()

            def _group(g, carry):
                for b in range(2):
                    ch = g * 2 + b
                    _gather(ch, b).wait()
                    _wload(ch, b).wait()

                    @pl.when(g > 0)
                    def _():
                        _scatter(ch - 2, b).wait()

                    def _scale(e4, inner):
                        e = e4 * 4
                        wss = [wbuf[b][pl.ds((e + k) * 16, 16)]
                               for k in range(4)]
                        for k in range(4):
                            for j in range(_DH // 16):
                                sl = pl.ds(j * 16, 16)
                                sbuf[b][e + k, sl] = gbuf[b][e + k, sl] * wss[k]
                        return inner

                    lax.fori_loop(0, _C // 4, _scale, 0)
                    _scatter(ch, b).start(add=True)

                    @pl.when(ch + 2 < ncw)
                    def _():
                        _gather(ch + 2, b).start()
                        _wload(ch + 2, b).start()
                return carry

            lax.fori_loop(0, ncw // 2, _group, 0)
            for b in range(2):
                _scatter(ncw - 2 + b, b).wait()
            plsc.subcore_barrier()

            def _lsl(t):
                return pl.ds(sid * rpt + t * _CZ, _CZ)

            def _gsl(t):
                return pl.ds(cid * Np + sid * rpt + t * _CZ, _CZ)

            def _racc(t, b):
                return pltpu.make_async_copy(acc.at[_lsl(t)], abuf[b], gsem[b])

            def _rsrc(t, b):
                return pltpu.make_async_copy(
                    accsrc.at[_gsl(t)], sbuf[b].at[pl.ds(0, _CZ)], ssem[b])

            def _wacc(t, b):
                return pltpu.make_async_copy(
                    sbuf[b].at[pl.ds(0, _CZ)], accout.at[_gsl(t)], wsem[b])

            def _wtab(t, b):
                return pltpu.make_async_copy(abuf[b], tab.at[_gsl(t)], wsem[b])

            for t in range(min(2, nz)):
                _racc(t, t).start()
                _rsrc(t, t).start()
            for t in range(nz):
                b = t % 2
                _racc(t, b).wait()
                _rsrc(t, b).wait()

                def _acc_upd(i, carry):
                    for j in range(_DH // 16):
                        sl = pl.ds(j * 16, 16)
                        sbuf[b][i, sl] = (sbuf[b][i, sl] + abuf[b][i, sl]) * scale
                    return carry

                lax.fori_loop(0, _CZ, _acc_upd, 0)
                _wacc(t, b).start()
                if tab is not None:
                    _wtab(t, b).start()
                if t + 2 < nz:
                    _wacc(t, b).wait()
                    if tab is not None:
                        _wtab(t, b).wait()
                    _racc(t + 2, b).start()
                    _rsrc(t + 2, b).start()
            for t in range(max(0, nz - 2), nz):
                b = t % 2
                _wacc(t, b).wait()
                if tab is not None:
                    _wtab(t, b).wait()

        srcs = [tab0, tabA, tabB]
        for l in range(n_layers):
            _phase(
                srcs[l],
                accin if l == 0 else accout,
                srcs[l + 1] if l + 1 < n_layers else None,
                1.0 / (n_layers + 1) if l == n_layers - 1 else 1.0,
            )

    return prop


@functools.lru_cache(maxsize=None)
def _make_split(Np, D):
    B = 1024
    while Np % B:
        B //= 2

    def body(x_ref, o_ref):
        o_ref[0] = x_ref[:, :_DH]
        o_ref[1] = x_ref[:, _DH:]

    return pl.pallas_call(
        body,
        grid=(Np // B,),
        in_specs=[pl.BlockSpec((B, D), lambda i: (i, 0))],
        out_specs=pl.BlockSpec((2, B, _DH), lambda i: (0, i, 0)),
        out_shape=jax.ShapeDtypeStruct((2, Np, _DH), jnp.float32),
    )


@functools.lru_cache(maxsize=None)
def _make_merge(Np, D):
    B = 1024
    while Np % B:
        B //= 2

    def body(x_ref, o_ref):
        o_ref[:, :_DH] = x_ref[0]
        o_ref[:, _DH:] = x_ref[1]

    return pl.pallas_call(
        body,
        grid=(Np // B,),
        in_specs=[pl.BlockSpec((2, B, _DH), lambda i: (0, i, 0))],
        out_specs=pl.BlockSpec((B, D), lambda i: (i, 0)),
        out_shape=jax.ShapeDtypeStruct((Np, D), jnp.float32),
    )


def kernel(user_emb_weight, item_emb_weight, edge_index, edge_weight):
    U, D = user_emb_weight.shape
    N = U + item_emb_weight.shape[0]
    E = edge_weight.shape[0]

    all_emb = jnp.concatenate([user_emb_weight, item_emb_weight], axis=0)

    Np = -(-N // (_NS * _C)) * (_NS * _C)
    emb = all_emb
    if Np > N:
        emb = jnp.concatenate(
            [emb, jnp.zeros((Np - N, D), jnp.float32)], axis=0)

    ncw = -(-E // (_NS * _C))
    ncw += ncw % 2
    Ep = _NS * _C * ncw
    row = edge_index[0].astype(jnp.int32)
    col = edge_index[1].astype(jnp.int32)
    w = edge_weight.astype(jnp.float32)
    if Ep > E:
        pad = Ep - E
        row = jnp.concatenate([row, jnp.zeros((pad,), jnp.int32)])
        col = jnp.concatenate([col, jnp.zeros((pad,), jnp.int32)])
        w = jnp.concatenate([w, jnp.zeros((pad,), jnp.float32)])
    col3 = col.reshape(_NS, ncw, _C)
    col4 = jnp.stack([col3, col3 + Np])
    row3 = row.reshape(_NS, ncw, _C)
    wexp = jnp.repeat(w, 16).reshape(_NS, ncw, _C * 16)

    emb2 = _make_split(Np, D)(emb).reshape(_NC * Np, _DH)
    _, _, acc = _make_prop(Np, ncw, 3)(emb2, emb2, col4, row3, wexp)
    out = _make_merge(Np, D)(acc.reshape(_NC, Np, _DH))[:N]
    return out[:U], out[U:]
